# baseline (speedup 1.0000x reference)
import functools

import jax
import jax.numpy as jnp
from jax import lax
from jax.experimental import pallas as pl
from jax.experimental.pallas import tpu as pltpu
from jax.experimental.pallas import tpu_sc as plsc


@functools.cache
def _make_sc_embed(B: int, S: int, V: int, D: int):
    info = plsc.get_sparse_core_info()
    NC, NS, L = info.num_cores, info.num_subcores, info.num_lanes
    NW = NC * NS
    assert S % NW == 0 and B % 2 == 0
    s_per_w = S // NW
    SUB = 16
    n_t = s_per_w // SUB
    n_pair = B // 2
    n_groups = n_t * n_pair
    NSLOT = 3
    mesh = plsc.VectorSubcoreMesh(core_axis_name="c", subcore_axis_name="s")

    @functools.partial(
        pl.kernel,
        mesh=mesh,
        out_type=jax.ShapeDtypeStruct((B * S, D), jnp.float32),
        scratch_types=[
            pltpu.VMEM((B, s_per_w), jnp.int32),
            pltpu.VMEM((n_groups * 2 * SUB,), jnp.int32),
            [pltpu.VMEM((2 * SUB, D), jnp.float32) for _ in range(NSLOT)],
            pltpu.VMEM((SUB, D), jnp.float32),
            [pltpu.SemaphoreType.DMA for _ in range(NSLOT)],
            [pltpu.SemaphoreType.DMA for _ in range(NSLOT)],
            pltpu.SemaphoreType.DMA,
            pltpu.SemaphoreType.DMA,
        ],
    )
    def emb(idx_hbm, table_hbm, pos_hbm, out_hbm,
            idx_v1, idx_v2, wbuf, pbuf, gsem, osem, psem, isem):
        wid = lax.axis_index("s") * NC + lax.axis_index("c")
        s_base = wid * s_per_w

        pltpu.async_copy(
            idx_hbm.at[:, pl.ds(s_base, s_per_w)], idx_v1, isem
        ).wait()

        for g in range(n_groups):
            t, p = divmod(g, n_pair)
            for i in range(2):
                v = idx_v1[2 * p + i, pl.ds(t * SUB, L)]
                idx_v2[pl.ds((g * 2 + i) * SUB, L)] = v

        def gather(g, slot):
            return pltpu.async_copy(
                table_hbm.at[idx_v2.at[pl.ds(g * 2 * SUB, 2 * SUB)]],
                wbuf[slot], gsem[slot],
            )

        def fill_pos(t):
            return pltpu.async_copy(
                pos_hbm.at[pl.ds(s_base + t * SUB, SUB)], pbuf, psem
            )

        def add_group(g, slot):
            pos = pbuf
            cur = wbuf[slot]

            def body(r, carry):
                for j in range(D // L):
                    sl = pl.ds(j * L, L)
                    pv = pos[r, sl]
                    plsc.addupdate(cur.at[r, sl], pv)
                    plsc.addupdate(cur.at[SUB + r, sl], pv)
                return carry

            lax.fori_loop(0, SUB, body, 0)

        def store_group(g, slot):
            t, p = divmod(g, n_pair)
            return [pltpu.async_copy(
                wbuf[slot].at[pl.ds(i * SUB, SUB)],
                out_hbm.at[pl.ds((2 * p + i) * S + s_base + t * SUB, SUB)],
                osem[slot],
            ) for i in range(2)]

        pend_pos = fill_pos(0)
        pend_g = [None] * NSLOT
        pend_o = [None] * NSLOT
        pend_g[0] = gather(0, 0)
        pend_g[1] = gather(1, 1)
        for g in range(n_groups):
            t, p = divmod(g, n_pair)
            slot = g % NSLOT
            if p == 0:
                pend_pos.wait()
            if g + 2 < n_groups:
                s2 = (g + 2) % NSLOT
                if pend_o[s2] is not None:
                    for d in pend_o[s2]:
                        d.wait()
                    pend_o[s2] = None
                pend_g[s2] = gather(g + 2, s2)
            pend_g[slot].wait()
            pend_g[slot] = None
            add_group(g, slot)
            if p == n_pair - 1 and t + 1 < n_t:
                pend_pos = fill_pos(t + 1)
            pend_o[slot] = store_group(g, slot)
        for descs in pend_o:
            if descs is not None:
                for d in descs:
                    d.wait()

    return emb


def kernel(input_ids, word_embeddings, position_embeddings):
    B, S = input_ids.shape
    V, D = word_embeddings.shape
    emb = _make_sc_embed(B, S, V, D)
    out = emb(input_ids.astype(jnp.int32), word_embeddings, position_embeddings)
    return out.reshape(B, S, D)


# speedup vs baseline: 1.1904x; 1.1904x over previous
import functools

import jax
import jax.numpy as jnp
from jax import lax
from jax.experimental import pallas as pl
from jax.experimental.pallas import tpu as pltpu
from jax.experimental.pallas import tpu_sc as plsc


@functools.cache
def _make_sc_embed(B: int, S: int, V: int, D: int):
    info = plsc.get_sparse_core_info()
    NC, NS, L = info.num_cores, info.num_subcores, info.num_lanes
    NW = NC * NS
    assert S % NW == 0
    s_per_w = S // NW
    CHUNK = 32
    assert s_per_w % CHUNK == 0
    n_sc = s_per_w // CHUNK
    n_steps = n_sc * B
    mesh = plsc.VectorSubcoreMesh(core_axis_name="c", subcore_axis_name="s")

    @functools.partial(
        pl.kernel,
        mesh=mesh,
        out_type=jax.ShapeDtypeStruct((B * S, D), jnp.float32),
        scratch_types=[
            pltpu.VMEM((B * s_per_w,), jnp.int32),
            pltpu.VMEM((CHUNK, D), jnp.float32),
            pltpu.VMEM((CHUNK, D), jnp.float32),
            pltpu.VMEM((CHUNK, D), jnp.float32),
            pltpu.SemaphoreType.DMA,
            pltpu.SemaphoreType.DMA,
            pltpu.SemaphoreType.DMA,
            pltpu.SemaphoreType.DMA,
            pltpu.SemaphoreType.DMA,
        ],
    )
    def emb(idx_hbm, table_hbm, pos_hbm, out_hbm,
            idx_v, w0, w1, pos_v, g0, g1, o0, o1, psem):
        wid = lax.axis_index("s") * NC + lax.axis_index("c")
        s_base = wid * s_per_w
        wbuf = (w0, w1)
        gsem = (g0, g1)
        osem = (o0, o1)

        for b in range(B):
            pltpu.sync_copy(
                idx_hbm.at[pl.ds(b * S + s_base, s_per_w)],
                idx_v.at[pl.ds(b * s_per_w, s_per_w)],
            )

        def gather(k, buf):
            sc, b = divmod(k, B)
            off = b * s_per_w + sc * CHUNK
            return pltpu.async_copy(
                table_hbm.at[idx_v.at[pl.ds(off, CHUNK)]],
                wbuf[buf], gsem[buf],
            )

        def fill_pos(sc):
            return pltpu.async_copy(
                pos_hbm.at[pl.ds(s_base + sc * CHUNK, CHUNK)], pos_v, psem
            )

        def add_pos(buf):
            cur = wbuf[buf]

            def body(r, carry):
                for j in range(D // L):
                    sl = pl.ds(j * L, L)
                    plsc.addupdate(cur.at[r, sl], pos_v[r, sl])
                return carry

            lax.fori_loop(0, CHUNK, body, 0)

        pend_pos = fill_pos(0)
        pending_g = gather(0, 0)
        pending_o = [None, None]
        for k in range(n_steps):
            sc, b = divmod(k, B)
            cur = k % 2
            nxt = (k + 1) % 2
            if b == 0:
                pend_pos.wait()
            if k + 1 < n_steps:
                if pending_o[nxt] is not None:
                    pending_o[nxt].wait()
                    pending_o[nxt] = None
                next_g = gather(k + 1, nxt)
            pending_g.wait()
            add_pos(cur)
            if b == B - 1 and sc + 1 < n_sc:
                pend_pos = fill_pos(sc + 1)
            pending_o[cur] = pltpu.async_copy(
                wbuf[cur],
                out_hbm.at[pl.ds(b * S + s_base + sc * CHUNK, CHUNK)],
                osem[cur],
            )
            if k + 1 < n_steps:
                pending_g = next_g
        for d in pending_o:
            if d is not None:
                d.wait()

    return emb


def kernel(input_ids, word_embeddings, position_embeddings):
    B, S = input_ids.shape
    V, D = word_embeddings.shape
    ids_flat = input_ids.reshape(-1).astype(jnp.int32)
    emb = _make_sc_embed(B, S, V, D)
    out = emb(ids_flat, word_embeddings, position_embeddings)
    return out.reshape(B, S, D)


# speedup vs baseline: 1.1922x; 1.0015x over previous
import functools

import jax
import jax.numpy as jnp
from jax import lax
from jax.experimental import pallas as pl
from jax.experimental.pallas import tpu as pltpu
from jax.experimental.pallas import tpu_sc as plsc


@functools.cache
def _make_sc_embed(B: int, S: int, V: int, D: int):
    info = plsc.get_sparse_core_info()
    NC, NS, L = info.num_cores, info.num_subcores, info.num_lanes
    NW = NC * NS
    assert S % NW == 0
    s_per_w = S // NW
    CHUNK = 32
    assert s_per_w % CHUNK == 0
    n_sc = s_per_w // CHUNK
    n_steps = n_sc * B
    mesh = plsc.VectorSubcoreMesh(core_axis_name="c", subcore_axis_name="s")

    @functools.partial(
        pl.kernel,
        mesh=mesh,
        out_type=jax.ShapeDtypeStruct((B * S, D), jnp.float32),
        scratch_types=[
            pltpu.VMEM((B * s_per_w,), jnp.int32),
            pltpu.VMEM((CHUNK, D), jnp.float32),
            pltpu.VMEM((CHUNK, D), jnp.float32),
            pltpu.VMEM((CHUNK, D), jnp.float32),
            pltpu.SemaphoreType.DMA,
            pltpu.SemaphoreType.DMA,
            pltpu.SemaphoreType.DMA,
            pltpu.SemaphoreType.DMA,
            pltpu.SemaphoreType.DMA,
        ],
    )
    def emb(idx_hbm, table_hbm, pos_hbm, out_hbm,
            idx_v, w0, w1, pos_v, g0, g1, o0, o1, psem):
        wid = lax.axis_index("s") * NC + lax.axis_index("c")
        s_base = wid * s_per_w
        wbuf = (w0, w1)
        gsem = (g0, g1)
        osem = (o0, o1)

        for b in range(B):
            pltpu.sync_copy(
                idx_hbm.at[pl.ds(b * S + s_base, s_per_w)],
                idx_v.at[pl.ds(b * s_per_w, s_per_w)],
            )

        def gather(k, buf):
            sc, b = divmod(k, B)
            off = b * s_per_w + sc * CHUNK
            return pltpu.async_copy(
                table_hbm.at[idx_v.at[pl.ds(off, CHUNK)]],
                wbuf[buf], gsem[buf],
            )

        def fill_pos(sc):
            return pltpu.async_copy(
                pos_hbm.at[pl.ds(s_base + sc * CHUNK, CHUNK)], pos_v, psem
            )

        def add_pos(buf):
            cur = wbuf[buf]

            def body(r, carry):
                for j in range(D // L):
                    sl = pl.ds(j * L, L)
                    plsc.addupdate(cur.at[r, sl], pos_v[r, sl])
                return carry

            lax.fori_loop(0, CHUNK, body, 0)

        pending_g = gather(0, 0)
        pending_o = [None, None]
        for k in range(n_steps):
            sc, b = divmod(k, B)
            cur = k % 2
            nxt = (k + 1) % 2
            if b == 0:
                fill_pos(sc).wait()
            if k + 1 < n_steps:
                if pending_o[nxt] is not None:
                    pending_o[nxt].wait()
                    pending_o[nxt] = None
                next_g = gather(k + 1, nxt)
            pending_g.wait()
            add_pos(cur)
            pending_o[cur] = pltpu.async_copy(
                wbuf[cur],
                out_hbm.at[pl.ds(b * S + s_base + sc * CHUNK, CHUNK)],
                osem[cur],
            )
            if k + 1 < n_steps:
                pending_g = next_g
        for d in pending_o:
            if d is not None:
                d.wait()

    return emb


def kernel(input_ids, word_embeddings, position_embeddings):
    B, S = input_ids.shape
    V, D = word_embeddings.shape
    ids_flat = input_ids.reshape(-1).astype(jnp.int32)
    emb = _make_sc_embed(B, S, V, D)
    out = emb(ids_flat, word_embeddings, position_embeddings)
    return out.reshape(B, S, D)


# speedup vs baseline: 1.2401x; 1.0402x over previous
import functools

import jax
import jax.numpy as jnp
from jax import lax
from jax.experimental import pallas as pl
from jax.experimental.pallas import tpu as pltpu
from jax.experimental.pallas import tpu_sc as plsc


@functools.cache
def _make_sc_embed(B: int, S: int, V: int, D: int):
    info = plsc.get_sparse_core_info()
    NC, NS, L = info.num_cores, info.num_subcores, info.num_lanes
    NW = NC * NS
    assert S % NW == 0
    s_per_w = S // NW
    CHUNK = 32
    assert s_per_w % CHUNK == 0
    n_sc = s_per_w // CHUNK
    n_steps = n_sc * B
    mesh = plsc.VectorSubcoreMesh(core_axis_name="c", subcore_axis_name="s")

    @functools.partial(
        pl.kernel,
        mesh=mesh,
        out_type=jax.ShapeDtypeStruct((B * S, D), jnp.float32),
        scratch_types=[
            pltpu.VMEM((B * s_per_w,), jnp.int32),
            pltpu.VMEM((CHUNK, D), jnp.float32),
            pltpu.VMEM((CHUNK, D), jnp.float32),
            pltpu.VMEM((CHUNK, D), jnp.float32),
            pltpu.SemaphoreType.DMA,
            pltpu.SemaphoreType.DMA,
            pltpu.SemaphoreType.DMA,
            pltpu.SemaphoreType.DMA,
            pltpu.SemaphoreType.DMA,
        ],
    )
    def emb(idx_hbm, table_hbm, pos_hbm, out_hbm,
            idx_v, w0, w1, pos_v, g0, g1, o0, o1, psem):
        wid = lax.axis_index("s") * NC + lax.axis_index("c")
        s_base = wid * s_per_w
        wbuf = (w0, w1)
        gsem = (g0, g1)
        osem = (o0, o1)

        for b in range(B):
            pltpu.sync_copy(
                idx_hbm.at[pl.ds(b * S + s_base, s_per_w)],
                idx_v.at[pl.ds(b * s_per_w, s_per_w)],
            )

        def gather(k, buf):
            sc, b = divmod(k, B)
            off = b * s_per_w + sc * CHUNK
            return pltpu.async_copy(
                table_hbm.at[idx_v.at[pl.ds(off, CHUNK)]],
                wbuf[buf], gsem[buf],
            )

        def fill_pos(sc):
            return pltpu.async_copy(
                pos_hbm.at[pl.ds(s_base + sc * CHUNK, CHUNK)], pos_v, psem
            )

        HALF = CHUNK // 2

        def add_half(buf, h):
            cur = wbuf[buf]

            def body(r, carry):
                for j in range(D // L):
                    sl = pl.ds(j * L, L)
                    plsc.addupdate(cur.at[r, sl], pos_v[r, sl])
                return carry

            lax.fori_loop(h * HALF, (h + 1) * HALF, body, 0)

        def store_half(k, buf, h):
            sc, b = divmod(k, B)
            return pltpu.async_copy(
                wbuf[buf].at[pl.ds(h * HALF, HALF)],
                out_hbm.at[pl.ds(b * S + s_base + sc * CHUNK + h * HALF, HALF)],
                osem[buf],
            )

        pend_pos = fill_pos(0)
        pending_g = gather(0, 0)
        pending_o = [[], []]
        for k in range(n_steps):
            sc, b = divmod(k, B)
            cur = k % 2
            nxt = (k + 1) % 2
            if b == 0:
                pend_pos.wait()
            if k + 1 < n_steps:
                for d in pending_o[nxt]:
                    d.wait()
                pending_o[nxt] = []
                next_g = gather(k + 1, nxt)
            pending_g.wait()
            add_half(cur, 0)
            pending_o[cur] = [store_half(k, cur, 0)]
            add_half(cur, 1)
            pending_o[cur].append(store_half(k, cur, 1))
            if b == B - 1 and sc + 1 < n_sc:
                pend_pos = fill_pos(sc + 1)
            if k + 1 < n_steps:
                pending_g = next_g
        for descs in pending_o:
            for d in descs:
                d.wait()

    return emb


def kernel(input_ids, word_embeddings, position_embeddings):
    B, S = input_ids.shape
    V, D = word_embeddings.shape
    ids_flat = input_ids.reshape(-1).astype(jnp.int32)
    emb = _make_sc_embed(B, S, V, D)
    out = emb(ids_flat, word_embeddings, position_embeddings)
    return out.reshape(B, S, D)
